# baseline (speedup 1.0000x reference)
import functools

import jax
import jax.numpy as jnp
from jax import lax
from jax.experimental import pallas as pl
from jax.experimental.pallas import tpu as pltpu
from jax.experimental.pallas import tpu_sc as plsc


def _tc_body(tab_ref, full_ref, out_ref):
    out_ref[...] = tab_ref[...][None, :, :]


def kernel(inputs, position_embeddings):
    batch, seq_len, d_model = inputs.shape
    sc_batches = batch // 2
    num_workers = 32
    rows_per_w = seq_len // num_workers
    chunk = 32
    n_chunks = rows_per_w // chunk
    mesh = plsc.VectorSubcoreMesh(core_axis_name="c", subcore_axis_name="s")

    @functools.partial(
        pl.kernel,
        mesh=mesh,
        out_type=jax.ShapeDtypeStruct((batch * seq_len, d_model), jnp.float32),
        scratch_types=[
            pltpu.VMEM((chunk, d_model), jnp.float32),
            pltpu.VMEM((chunk, d_model), jnp.float32),
            pltpu.SemaphoreType.DMA,
            pltpu.SemaphoreType.DMA,
            pltpu.SemaphoreType.DMA,
            pltpu.SemaphoreType.DMA,
        ],
    )
    def sc_copy(table_hbm, out_hbm, buf0, buf1, rsem0, rsem1, wsem0, wsem1):
        wid = lax.axis_index("s") * 2 + lax.axis_index("c")
        base = wid * rows_per_w
        bufs = (buf0, buf1)
        rsems = (rsem0, rsem1)
        wsems = (wsem0, wsem1)
        reads = [
            pltpu.async_copy(
                table_hbm.at[pl.ds(base + c * chunk, chunk)], bufs[c % 2], rsems[c % 2]
            )
            if c < 2
            else None
            for c in range(n_chunks)
        ]
        for c in range(n_chunks):
            reads[c].wait()
            writes = [
                pltpu.async_copy(
                    bufs[c % 2],
                    out_hbm.at[pl.ds(b * seq_len + base + c * chunk, chunk)],
                    wsems[c % 2],
                )
                for b in range(sc_batches)
            ]
            for w in writes:
                w.wait()
            if c + 2 < n_chunks:
                reads[c + 2] = pltpu.async_copy(
                    table_hbm.at[pl.ds(base + (c + 2) * chunk, chunk)],
                    bufs[c % 2],
                    rsems[c % 2],
                )

    full = sc_copy(position_embeddings).reshape(batch, seq_len, d_model)

    block_s = 512
    out = pl.pallas_call(
        _tc_body,
        grid=(seq_len // block_s, batch - sc_batches),
        in_specs=[
            pl.BlockSpec((block_s, d_model), lambda i, b: (i, 0)),
            pl.BlockSpec(memory_space=pltpu.MemorySpace.HBM),
        ],
        out_specs=pl.BlockSpec(
            (1, block_s, d_model), lambda i, b: (b + sc_batches, i, 0)
        ),
        out_shape=jax.ShapeDtypeStruct((batch, seq_len, d_model), position_embeddings.dtype),
        input_output_aliases={1: 0},
    )(position_embeddings, full)
    return out


# speedup vs baseline: 1.2901x; 1.2901x over previous
import functools

import jax
import jax.numpy as jnp
from jax import lax
from jax.experimental import pallas as pl
from jax.experimental.pallas import tpu as pltpu
from jax.experimental.pallas import tpu_sc as plsc


def kernel(inputs, position_embeddings):
    batch, seq_len, d_model = inputs.shape
    num_workers = 32
    rows_per_w = seq_len // num_workers
    chunk = 32
    n_chunks = rows_per_w // chunk
    nbuf = 3
    mesh = plsc.VectorSubcoreMesh(core_axis_name="c", subcore_axis_name="s")

    @functools.partial(
        pl.kernel,
        mesh=mesh,
        out_type=jax.ShapeDtypeStruct((batch * seq_len, d_model), jnp.float32),
        scratch_types=(
            [pltpu.VMEM((chunk, d_model), jnp.float32)] * nbuf
            + [pltpu.SemaphoreType.DMA] * (2 * nbuf)
        ),
    )
    def sc_copy(table_hbm, out_hbm, *refs):
        bufs = refs[:nbuf]
        rsems = refs[nbuf : 2 * nbuf]
        wsems = refs[2 * nbuf :]
        wid = lax.axis_index("s") * 2 + lax.axis_index("c")
        base = wid * rows_per_w

        def read(c):
            return pltpu.async_copy(
                table_hbm.at[pl.ds(base + c * chunk, chunk)],
                bufs[c % nbuf],
                rsems[c % nbuf],
            )

        reads = [read(c) if c < nbuf else None for c in range(n_chunks)]
        writes = [None] * n_chunks
        for c in range(n_chunks):
            reads[c].wait()
            writes[c] = [
                pltpu.async_copy(
                    bufs[c % nbuf],
                    out_hbm.at[pl.ds(b * seq_len + base + c * chunk, chunk)],
                    wsems[c % nbuf],
                )
                for b in range(batch)
            ]
            nxt = c + 1
            if nbuf <= nxt < n_chunks:
                for w in writes[nxt - nbuf]:
                    w.wait()
                reads[nxt] = read(nxt)
        for c in range(max(0, n_chunks - nbuf), n_chunks):
            for w in writes[c]:
                w.wait()

    out = sc_copy(position_embeddings)
    return out.reshape(batch, seq_len, d_model)


# speedup vs baseline: 2.2716x; 1.7608x over previous
import jax
import jax.numpy as jnp
from jax.experimental import pallas as pl


def _bcast_body(tab_ref, out_ref):
    out_ref[...] = jnp.broadcast_to(tab_ref[...][None, :, :], out_ref.shape)


def kernel(inputs, position_embeddings):
    batch, seq_len, d_model = inputs.shape
    block_s = 512
    grid = (seq_len // block_s,)
    out = pl.pallas_call(
        _bcast_body,
        grid=grid,
        in_specs=[
            pl.BlockSpec((block_s, d_model), lambda i: (i, 0)),
        ],
        out_specs=pl.BlockSpec((batch, block_s, d_model), lambda i: (0, i, 0)),
        out_shape=jax.ShapeDtypeStruct((batch, seq_len, d_model), position_embeddings.dtype),
    )(position_embeddings)
    return out
